# baseline (speedup 1.0000x reference)
import functools

import jax
import jax.numpy as jnp
from jax import lax
from jax.experimental import pallas as pl
from jax.experimental.pallas import tpu as pltpu
from jax.experimental.pallas import tpu_sc as plsc

BATCH = 16384
HIDDEN = 64
NUM_ROWS = 1000000
NUM_CORES = 2
NUM_SUBCORES = 16
NW = NUM_CORES * NUM_SUBCORES
B_PER_W = BATCH // NW
LANES = 16
CH = 128
N_CH = B_PER_W // CH


def _gather_body(tab_u, tab_i, user_ids, item_ids, uout, iout,
                 idx_v, rows_v, gsem):
  wid = lax.axis_index("s") * NUM_CORES + lax.axis_index("c")
  base = wid * B_PER_W
  for tab, ids_hbm, out in ((tab_u, user_ids, uout), (tab_i, item_ids, iout)):
    pltpu.sync_copy(ids_hbm.at[pl.ds(base, B_PER_W)], idx_v)
    for c in range(N_CH):
      pltpu.async_copy(
          tab.at[idx_v.at[pl.ds(c * CH, CH)]], rows_v, gsem).wait()
      pltpu.sync_copy(rows_v, out.at[pl.ds(base + c * CH, CH)])


@jax.jit
def _sc_gather(user_ids, item_ids, tab_u, tab_i):
  mesh = plsc.VectorSubcoreMesh(core_axis_name="c", subcore_axis_name="s")
  f = pl.kernel(
      _gather_body,
      mesh=mesh,
      out_type=(
          jax.ShapeDtypeStruct((BATCH, HIDDEN), jnp.bfloat16),
          jax.ShapeDtypeStruct((BATCH, HIDDEN), jnp.bfloat16),
      ),
      scratch_types=[
          pltpu.VMEM((B_PER_W,), jnp.int32),
          pltpu.VMEM((CH, HIDDEN), jnp.bfloat16),
          pltpu.SemaphoreType.DMA,
      ],
      compiler_params=pltpu.CompilerParams(
          skip_device_barrier=True, use_tc_tiling_on_sc=False),
  )
  return f(tab_u, tab_i, user_ids, item_ids)


def _mlp_body(u_ref, i_ref, w1a_ref, w1b_ref, b1_ref, w2_ref, b2_ref, o_ref):
  u = u_ref[...].astype(jnp.float32)
  it = i_ref[...].astype(jnp.float32)
  h = jnp.dot(u, w1a_ref[...], preferred_element_type=jnp.float32)
  h = h + jnp.dot(it, w1b_ref[...], preferred_element_type=jnp.float32)
  h = jnp.maximum(h + b1_ref[...], 0.0)
  logits = jnp.sum(h * w2_ref[...], axis=1, keepdims=True) + b2_ref[0, 0]
  o_ref[...] = 1.0 / (1.0 + jnp.exp(-logits))


@jax.jit
def _tc_mlp(u_emb, i_emb, W1, b1, W2, b2):
  w1a = W1[:HIDDEN]
  w1b = W1[HIDDEN:]
  b1r = b1.reshape(1, HIDDEN)
  w2r = W2.reshape(1, HIDDEN)
  b2r = b2.reshape(1, 1)
  RB = 2048
  grid = BATCH // RB
  return pl.pallas_call(
      _mlp_body,
      grid=(grid,),
      in_specs=[
          pl.BlockSpec((RB, HIDDEN), lambda g: (g, 0)),
          pl.BlockSpec((RB, HIDDEN), lambda g: (g, 0)),
          pl.BlockSpec((HIDDEN, HIDDEN), lambda g: (0, 0)),
          pl.BlockSpec((HIDDEN, HIDDEN), lambda g: (0, 0)),
          pl.BlockSpec((1, HIDDEN), lambda g: (0, 0)),
          pl.BlockSpec((1, HIDDEN), lambda g: (0, 0)),
          pl.BlockSpec((1, 1), lambda g: (0, 0)),
      ],
      out_specs=pl.BlockSpec((RB, 1), lambda g: (g, 0)),
      out_shape=jax.ShapeDtypeStruct((BATCH, 1), jnp.float32),
  )(u_emb, i_emb, w1a, w1b, b1r, w2r, b2r)


def kernel(user_ids, item_ids, user_table, item_table, W1, b1, W2, b2):
  tab_u = user_table.astype(jnp.bfloat16)
  tab_i = item_table.astype(jnp.bfloat16)
  u_emb, i_emb = _sc_gather(user_ids, item_ids, tab_u, tab_i)
  return _tc_mlp(u_emb, i_emb, W1, b1, W2, b2)


# speedup vs baseline: 2.0539x; 2.0539x over previous
import functools

import jax
import jax.numpy as jnp
from jax import lax
from jax.experimental import pallas as pl
from jax.experimental.pallas import tpu as pltpu
from jax.experimental.pallas import tpu_sc as plsc

BATCH = 16384
HIDDEN = 64
NUM_ROWS = 1000000
NUM_CORES = 2
NUM_SUBCORES = 16
NW = NUM_CORES * NUM_SUBCORES
B_PER_W = BATCH // NW
LANES = 16
N_SEM = 4


def _gather_body(tab_u, tab_i, user_ids, item_ids, uout, iout,
                 idx_v, rows_v, *sems):
  wid = lax.axis_index("s") * NUM_CORES + lax.axis_index("c")
  base = wid * B_PER_W
  for tab, ids_hbm, out in ((tab_u, user_ids, uout), (tab_i, item_ids, iout)):
    pltpu.sync_copy(ids_hbm.at[pl.ds(base, B_PER_W)], idx_v)

    def _issue(g, carry, tab=tab):
      ids = idx_v[pl.ds(g * LANES, LANES)]
      for j in range(LANES):
        k = g * LANES + j
        pltpu.async_copy(tab.at[pl.ds(ids[j], 1)],
                         rows_v.at[pl.ds(k, 1)],
                         sems[j % N_SEM])
      return carry

    lax.fori_loop(0, B_PER_W // LANES, _issue, 0)
    per_sem = B_PER_W // N_SEM
    for q in range(N_SEM):
      pltpu.make_async_copy(
          tab.at[pl.ds(0, per_sem)], rows_v.at[pl.ds(0, per_sem)],
          sems[q]).wait()
    pltpu.sync_copy(rows_v, out.at[pl.ds(base, B_PER_W)])


@jax.jit
def _sc_gather(user_ids, item_ids, user_table, item_table):
  mesh = plsc.VectorSubcoreMesh(core_axis_name="c", subcore_axis_name="s")
  f = pl.kernel(
      _gather_body,
      mesh=mesh,
      out_type=(
          jax.ShapeDtypeStruct((BATCH, HIDDEN), jnp.float32),
          jax.ShapeDtypeStruct((BATCH, HIDDEN), jnp.float32),
      ),
      scratch_types=[
          pltpu.VMEM((B_PER_W,), jnp.int32),
          pltpu.VMEM((B_PER_W, HIDDEN), jnp.float32),
      ] + [pltpu.SemaphoreType.DMA] * N_SEM,
      compiler_params=pltpu.CompilerParams(skip_device_barrier=True),
  )
  return f(user_table, item_table, user_ids, item_ids)


def _mlp_body(u_ref, i_ref, w1a_ref, w1b_ref, b1_ref, w2_ref, b2_ref, o_ref):
  u = u_ref[...]
  it = i_ref[...]
  h = jnp.dot(u, w1a_ref[...], preferred_element_type=jnp.float32)
  h = h + jnp.dot(it, w1b_ref[...], preferred_element_type=jnp.float32)
  h = jnp.maximum(h + b1_ref[...], 0.0)
  logits = jnp.sum(h * w2_ref[...], axis=1, keepdims=True) + b2_ref[0, 0]
  o_ref[...] = 1.0 / (1.0 + jnp.exp(-logits))


@jax.jit
def _tc_mlp(u_emb, i_emb, W1, b1, W2, b2):
  w1a = W1[:HIDDEN]
  w1b = W1[HIDDEN:]
  b1r = b1.reshape(1, HIDDEN)
  w2r = W2.reshape(1, HIDDEN)
  b2r = b2.reshape(1, 1)
  RB = 2048
  grid = BATCH // RB
  return pl.pallas_call(
      _mlp_body,
      grid=(grid,),
      in_specs=[
          pl.BlockSpec((RB, HIDDEN), lambda g: (g, 0)),
          pl.BlockSpec((RB, HIDDEN), lambda g: (g, 0)),
          pl.BlockSpec((HIDDEN, HIDDEN), lambda g: (0, 0)),
          pl.BlockSpec((HIDDEN, HIDDEN), lambda g: (0, 0)),
          pl.BlockSpec((1, HIDDEN), lambda g: (0, 0)),
          pl.BlockSpec((1, HIDDEN), lambda g: (0, 0)),
          pl.BlockSpec((1, 1), lambda g: (0, 0)),
      ],
      out_specs=pl.BlockSpec((RB, 1), lambda g: (g, 0)),
      out_shape=jax.ShapeDtypeStruct((BATCH, 1), jnp.float32),
  )(u_emb, i_emb, w1a, w1b, b1r, w2r, b2r)


def kernel(user_ids, item_ids, user_table, item_table, W1, b1, W2, b2):
  u_emb, i_emb = _sc_gather(user_ids, item_ids, user_table, item_table)
  return _tc_mlp(u_emb, i_emb, W1, b1, W2, b2)
